# baseline (speedup 1.0000x reference)
import functools

import jax
import jax.numpy as jnp
from jax import lax
from jax.experimental import pallas as pl
from jax.experimental.pallas import tpu as pltpu
from jax.experimental.pallas import tpu_sc as plsc

_N_SC = 2
_N_SUB = 16
_NW = _N_SC * _N_SUB
_B = 40


def _silu(x):
    return x * jax.nn.sigmoid(x)


def _mlp1_body(s_ref, v_ref, Wi1_ref, bi1_ref, Wi2_ref, bi2_ref,
               x0_ref, xy0_ref, xy1_ref, xy2_ref):
    h = s_ref.shape[1]
    hh = jnp.dot(s_ref[...], Wi1_ref[...], preferred_element_type=jnp.float32)
    hh = _silu(hh + bi1_ref[...])
    x = jnp.dot(hh, Wi2_ref[...], preferred_element_type=jnp.float32) + bi2_ref[...]
    x0_ref[...] = x[:, :h]
    x1 = x[:, h:2 * h]
    x2 = x[:, 2 * h:]
    xy0_ref[...] = jnp.concatenate([x1, x2 * v_ref[:, 0, :]], axis=1)
    xy1_ref[...] = jnp.concatenate([x1, x2 * v_ref[:, 1, :]], axis=1)
    xy2_ref[...] = jnp.concatenate([x1, x2 * v_ref[:, 2, :]], axis=1)


def _node_mlp(s, v, Wi1, bi1, Wi2, bi2, block=1000):
    n, h = s.shape
    h3 = Wi2.shape[1]
    grid = (n // block,)
    out_sp = pl.BlockSpec((block, h), lambda i: (i, 0))
    return pl.pallas_call(
        _mlp1_body,
        grid=grid,
        in_specs=[
            pl.BlockSpec((block, h), lambda i: (i, 0)),
            pl.BlockSpec((block, 3, h), lambda i: (i, 0, 0)),
            pl.BlockSpec((h, h), lambda i: (0, 0)),
            pl.BlockSpec((h,), lambda i: (0,)),
            pl.BlockSpec((h, h3), lambda i: (0, 0)),
            pl.BlockSpec((h3,), lambda i: (0,)),
        ],
        out_specs=(out_sp,) + (pl.BlockSpec((block, 2 * h), lambda i: (i, 0)),) * 3,
        out_shape=(jax.ShapeDtypeStruct((n, h), jnp.float32),)
        + tuple(jax.ShapeDtypeStruct((n, 2 * h), jnp.float32) for _ in range(3)),
    )(s, v, Wi1, bi1, Wi2, bi2)


def _sc_pipeline(nblk, g, nsup, run_issue, run_wait, compute_tmpl, idxb,
                 idx2h, wid, msg_src, acc, semd, sems, semi):
    pass


def _sc_edge_stage(x0, xy0, xy1, xy2, W0, Wd0, Wd1, Wd2, W2, idx2, zeros,
                   which):
    n, h = x0.shape
    nblk = W0.shape[1]
    b = W0.shape[2]
    nsup = idx2.shape[1]
    g = idx2.shape[3]
    rows = (n // _N_SUB) // 8 * 8
    tail = n - _N_SUB * rows
    nout = 1 if which == 0 else 3
    mesh = plsc.VectorSubcoreMesh(core_axis_name="c", subcore_axis_name="s")

    @functools.partial(
        pl.kernel,
        mesh=mesh,
        out_type=jax.ShapeDtypeStruct((nout, _N_SC, n, h), jnp.float32),
        scratch_types=[
            pltpu.VMEM((2, 2, g, b), jnp.int32),
            pltpu.VMEM((2, b, h), jnp.float32),
            pltpu.VMEM((2, b, h), jnp.float32),
            pltpu.VMEM((2, b, 2 * h), jnp.float32),
            pltpu.VMEM_SHARED((n, h), jnp.float32),
            pltpu.SemaphoreType.DMA,
            pltpu.SemaphoreType.DMA,
            pltpu.SemaphoreType.DMA,
        ],
    )
    def k(x0h, xy0h, xy1h, xy2h, w0h, wd0h, wd1h, wd2h, w2h, idx2h, zh,
          outh, idxb, wa2, wb2, gxy2, acc, semd, sems, semi):
        cid = lax.axis_index("c")
        sid = lax.axis_index("s")
        wid = sid * _N_SC + cid
        row0 = sid * rows
        xytabs = (xy0h, xy1h, xy2h)
        nchunk = h // 16

        def wait_scatter():
            pltpu.make_async_copy(wa2.at[0], acc.at[idxb.at[0, 1, 0]], sems).wait()

        def run_pass(issue, waitd, compute):
            pltpu.sync_copy(idx2h.at[wid, 0], idxb.at[0])
            issue(0, 0, 0, 0)

            def body(i, carry):
                p = lax.rem(i, 2)
                sup = lax.div(i, g)
                j = i - sup * g
                q = lax.rem(sup, 2)
                waitd(i, p)

                @pl.when(i > 0)
                def _():
                    wait_scatter()

                @pl.when(jnp.logical_and(j == 0, sup + 1 < nsup))
                def _():
                    pltpu.async_copy(idx2h.at[wid, sup + 1],
                                     idxb.at[lax.rem(sup + 1, 2)], semi)

                i1 = i + 1

                @pl.when(i1 < nblk)
                def _():
                    @pl.when(lax.rem(i1, g) == 0)
                    def _():
                        pltpu.make_async_copy(idx2h.at[wid, 0], idxb.at[0],
                                              semi).wait()
                    sup1 = lax.div(i1, g)
                    issue(i1, lax.rem(i1, 2), lax.rem(sup1, 2), i1 - sup1 * g)

                compute(p)
                pltpu.async_copy(wa2.at[p], acc.at[idxb.at[q, 1, j]], sems,
                                 add=True)
                return carry

            lax.fori_loop(0, nblk, body, 0)
            wait_scatter()

        chunks = (0,) if which == 0 else (1, 2, 3)
        for ci, chunk in enumerate(chunks):
            pltpu.sync_copy(zh.at[pl.ds(row0, rows)], acc.at[pl.ds(row0, rows)])
            if tail:
                @pl.when(sid == _N_SUB - 1)
                def _():
                    pltpu.sync_copy(zh.at[pl.ds(_N_SUB * rows, tail)],
                                    acc.at[pl.ds(_N_SUB * rows, tail)])
            plsc.subcore_barrier()
            if chunk == 0:
                def issue0(i, p, q, j):
                    pltpu.async_copy(w0h.at[wid, i], wa2.at[p], semd)
                    pltpu.async_copy(x0h.at[idxb.at[q, 0, j]], wb2.at[p], semd)

                def waitd0(i, p):
                    pltpu.make_async_copy(w0h.at[wid, 0], wa2.at[p], semd).wait()
                    pltpu.make_async_copy(x0h.at[idxb.at[0, 0, 0]], wb2.at[p],
                                          semd).wait()

                def compute0(p):
                    def row(bb, cc):
                        for kk in range(nchunk):
                            sl = pl.ds(kk * 16, 16)
                            wa2[p, bb, sl] = wa2[p, bb, sl] * wb2[p, bb, sl]
                        return cc

                    lax.fori_loop(0, b, row, 0)

                run_pass(issue0, waitd0, compute0)
            else:
                xyh = xytabs[chunk - 1]
                wdh = (wd0h, wd1h, wd2h)[chunk - 1]

                def issue1(i, p, q, j):
                    pltpu.async_copy(wdh.at[wid, i], wa2.at[p], semd)
                    pltpu.async_copy(w2h.at[wid, i], wb2.at[p], semd)
                    pltpu.async_copy(xyh.at[idxb.at[q, 0, j]], gxy2.at[p], semd)

                def waitd1(i, p):
                    pltpu.make_async_copy(wdh.at[wid, 0], wa2.at[p], semd).wait()
                    pltpu.make_async_copy(w2h.at[wid, 0], wb2.at[p], semd).wait()
                    pltpu.make_async_copy(xyh.at[idxb.at[0, 0, 0]], gxy2.at[p],
                                          semd).wait()

                def compute1(p):
                    def row(bb, cc):
                        for kk in range(nchunk):
                            sl = pl.ds(kk * 16, 16)
                            slr = pl.ds(h + kk * 16, 16)
                            wa2[p, bb, sl] = (wa2[p, bb, sl] * gxy2[p, bb, sl]
                                              + wb2[p, bb, sl] * gxy2[p, bb, slr])
                        return cc

                    lax.fori_loop(0, b, row, 0)

                run_pass(issue1, waitd1, compute1)
            plsc.subcore_barrier()
            pltpu.sync_copy(acc.at[pl.ds(row0, rows)],
                            outh.at[ci, cid, pl.ds(row0, rows)])
            if tail:
                @pl.when(sid == _N_SUB - 1)
                def _():
                    pltpu.sync_copy(acc.at[pl.ds(_N_SUB * rows, tail)],
                                    outh.at[ci, cid, pl.ds(_N_SUB * rows, tail)])
            plsc.subcore_barrier()

    return k(x0, xy0, xy1, xy2, W0, Wd0, Wd1, Wd2, W2, idx2, zeros)


def _update_body(s_ref, v_ref, ep0_ref, ep1_ref, Wv_ref, Wm1_ref, bm1_ref,
                 Wm2_ref, bm2_ref, s_out_ref, v_out_ref):
    h = s_ref.shape[1]
    s1 = s_ref[...] + jnp.clip(ep0_ref[0, 0] + ep0_ref[0, 1], -10000.0, 10000.0)
    v1 = [v_ref[:, k, :] + jnp.clip(ep1_ref[k, 0] + ep1_ref[k, 1],
                                    -10000.0, 10000.0) for k in range(3)]
    vl = []
    vr = []
    for k in range(3):
        vmk = jnp.dot(v1[k], Wv_ref[...], preferred_element_type=jnp.float32)
        vl.append(vmk[:, :h])
        vr.append(vmk[:, h:])
    v_norm = jnp.sqrt(vl[0] ** 2 + vl[1] ** 2 + vl[2] ** 2 + 1e-08)
    ts = jnp.concatenate([s1, v_norm], axis=-1)
    m = jnp.dot(ts, Wm1_ref[...], preferred_element_type=jnp.float32) + bm1_ref[...]
    m = jnp.dot(_silu(m), Wm2_ref[...], preferred_element_type=jnp.float32) + bm2_ref[...]
    ds2 = m[:, :h]
    dvm = m[:, h:2 * h]
    dsv = m[:, 2 * h:]
    dsv = dsv * (vl[0] * vr[0] + vl[1] * vr[1] + vl[2] * vr[2])
    s_out_ref[...] = s1 + jnp.clip(ds2 + dsv, -10000.0, 10000.0)
    dvm3 = jnp.stack([v1[k] + jnp.clip(dvm * vr[k], -10000.0, 10000.0)
                      for k in range(3)], axis=1)
    v_out_ref[...] = dvm3


def _update_block(s, v, ep0, ep1, Wv, Wm1, bm1, Wm2, bm2, block=1000):
    n, h = s.shape
    grid = (n // block,)
    out_shapes = (
        jax.ShapeDtypeStruct((n, h), jnp.float32),
        jax.ShapeDtypeStruct((n, 3, h), jnp.float32),
    )
    return pl.pallas_call(
        _update_body,
        grid=grid,
        in_specs=[
            pl.BlockSpec((block, h), lambda i: (i, 0)),
            pl.BlockSpec((block, 3, h), lambda i: (i, 0, 0)),
            pl.BlockSpec((1, 2, block, h), lambda i: (0, 0, i, 0)),
            pl.BlockSpec((3, 2, block, h), lambda i: (0, 0, i, 0)),
            pl.BlockSpec(Wv.shape, lambda i: (0, 0)),
            pl.BlockSpec(Wm1.shape, lambda i: (0, 0)),
            pl.BlockSpec(bm1.shape, lambda i: (0,)),
            pl.BlockSpec(Wm2.shape, lambda i: (0, 0)),
            pl.BlockSpec(bm2.shape, lambda i: (0,)),
        ],
        out_specs=(
            pl.BlockSpec((block, h), lambda i: (i, 0)),
            pl.BlockSpec((block, 3, h), lambda i: (i, 0, 0)),
        ),
        out_shape=out_shapes,
    )(s, v, ep0, ep1, Wv, Wm1, bm1, Wm2, bm2)


def kernel(s, v, dir_ij, Wij, senders, receivers, Wi1, bi1, Wi2, bi2, Wm1, bm1, Wm2, bm2, Wv):
    n, h = s.shape
    e = senders.shape[0]
    epw = e // _NW
    nblk = epw // _B
    x0, xy0, xy1, xy2 = _node_mlp(s, v, Wi1, bi1, Wi2, bi2)
    W0, W1, W2 = jnp.split(Wij, 3, axis=1)
    W0 = W0.reshape(_NW, nblk, _B, h)
    Wd0 = (W1 * dir_ij[:, 0:1]).reshape(_NW, nblk, _B, h)
    Wd1 = (W1 * dir_ij[:, 1:2]).reshape(_NW, nblk, _B, h)
    Wd2 = (W1 * dir_ij[:, 2:3]).reshape(_NW, nblk, _B, h)
    W2 = W2.reshape(_NW, nblk, _B, h)
    g = 10
    nsup = nblk // g
    recv4 = receivers.reshape(_NW, nsup, g, _B)
    send4 = senders.reshape(_NW, nsup, g, _B)
    idx2 = jnp.stack([recv4, send4], axis=2)
    zeros = jnp.zeros((n, h), jnp.float32)
    ep0 = _sc_edge_stage(x0, xy0, xy1, xy2, W0, Wd0, Wd1, Wd2, W2, idx2, zeros,
                         which=0)
    ep1 = _sc_edge_stage(x0, xy0, xy1, xy2, W0, Wd0, Wd1, Wd2, W2, idx2, zeros,
                         which=1)
    return _update_block(s, v, ep0, ep1, Wv, Wm1, bm1, Wm2, bm2)


# speedup vs baseline: 1.0012x; 1.0012x over previous
import functools

import jax
import jax.numpy as jnp
from jax import lax
from jax.experimental import pallas as pl
from jax.experimental.pallas import tpu as pltpu
from jax.experimental.pallas import tpu_sc as plsc

_N_SC = 2
_N_SUB = 16
_NW = _N_SC * _N_SUB
_B = 40


def _silu(x):
    return x * jax.nn.sigmoid(x)


def _mlp1_body(s_ref, v_ref, Wi1_ref, bi1_ref, Wi2_ref, bi2_ref,
               x0_ref, xy0_ref, xy1_ref, xy2_ref):
    h = s_ref.shape[1]
    hh = jnp.dot(s_ref[...], Wi1_ref[...], preferred_element_type=jnp.float32)
    hh = _silu(hh + bi1_ref[...])
    x = jnp.dot(hh, Wi2_ref[...], preferred_element_type=jnp.float32) + bi2_ref[...]
    x0_ref[...] = x[:, :h]
    x1 = x[:, h:2 * h]
    x2 = x[:, 2 * h:]
    xy0_ref[...] = jnp.concatenate([x1, x2 * v_ref[:, 0, :]], axis=1)
    xy1_ref[...] = jnp.concatenate([x1, x2 * v_ref[:, 1, :]], axis=1)
    xy2_ref[...] = jnp.concatenate([x1, x2 * v_ref[:, 2, :]], axis=1)


def _node_mlp(s, v, Wi1, bi1, Wi2, bi2, block=1000):
    n, h = s.shape
    h3 = Wi2.shape[1]
    grid = (n // block,)
    out_sp = pl.BlockSpec((block, h), lambda i: (i, 0))
    return pl.pallas_call(
        _mlp1_body,
        grid=grid,
        in_specs=[
            pl.BlockSpec((block, h), lambda i: (i, 0)),
            pl.BlockSpec((block, 3, h), lambda i: (i, 0, 0)),
            pl.BlockSpec((h, h), lambda i: (0, 0)),
            pl.BlockSpec((h,), lambda i: (0,)),
            pl.BlockSpec((h, h3), lambda i: (0, 0)),
            pl.BlockSpec((h3,), lambda i: (0,)),
        ],
        out_specs=(out_sp,) + (pl.BlockSpec((block, 2 * h), lambda i: (i, 0)),) * 3,
        out_shape=(jax.ShapeDtypeStruct((n, h), jnp.float32),)
        + tuple(jax.ShapeDtypeStruct((n, 2 * h), jnp.float32) for _ in range(3)),
    )(s, v, Wi1, bi1, Wi2, bi2)


def _sc_edge_stage(x0, xy0, xy1, xy2, W0, Wd0, Wd1, Wd2, W2, idx2, zeros):
    n, h = x0.shape
    nblk = W0.shape[1]
    b = W0.shape[2]
    nsup = idx2.shape[1]
    g = idx2.shape[3]
    rows = (n // _N_SUB) // 8 * 8
    tail = n - _N_SUB * rows
    mesh = plsc.VectorSubcoreMesh(core_axis_name="c", subcore_axis_name="s")

    @functools.partial(
        pl.kernel,
        mesh=mesh,
        out_type=jax.ShapeDtypeStruct((4, _N_SC, n, h), jnp.float32),
        scratch_types=[
            pltpu.VMEM((2, 2, g, b), jnp.int32),
            pltpu.VMEM((2, b, h), jnp.float32),
            pltpu.VMEM((2, b, h), jnp.float32),
            pltpu.VMEM((2, b, 2 * h), jnp.float32),
            pltpu.VMEM_SHARED((n, h), jnp.float32),
            pltpu.SemaphoreType.DMA,
            pltpu.SemaphoreType.DMA,
            pltpu.SemaphoreType.DMA,
        ],
    )
    def k(x0h, xy0h, xy1h, xy2h, w0h, wd0h, wd1h, wd2h, w2h, idx2h, zh,
          outh, idxb, wa2, wb2, gxy2, acc, semd, sems, semi):
        cid = lax.axis_index("c")
        sid = lax.axis_index("s")
        wid = sid * _N_SC + cid
        row0 = sid * rows
        xytabs = (xy0h, xy1h, xy2h)
        nchunk = h // 16

        def wait_scatter():
            pltpu.make_async_copy(wa2.at[0], acc.at[idxb.at[0, 1, 0]], sems).wait()

        def wait_idx():
            pltpu.make_async_copy(idx2h.at[wid, 0], idxb.at[0], semi).wait()

        def run_pass(issue, waitd, compute):
            pltpu.sync_copy(idx2h.at[wid, 0], idxb.at[0])
            issue(0, 0, 0, 0)

            def body(i, carry):
                p = lax.rem(i, 2)
                sup = lax.div(i, g)
                j = i - sup * g
                q = lax.rem(sup, 2)
                waitd(i, p)

                @pl.when(i > 0)
                def _():
                    wait_scatter()

                @pl.when(jnp.logical_and(j == 0, sup + 1 < nsup))
                def _():
                    pltpu.async_copy(idx2h.at[wid, sup + 1],
                                     idxb.at[lax.rem(sup + 1, 2)], semi)

                i1 = i + 1

                @pl.when(i1 < nblk)
                def _():
                    @pl.when(lax.rem(i1, g) == 0)
                    def _():
                        wait_idx()
                    sup1 = lax.div(i1, g)
                    issue(i1, lax.rem(i1, 2), lax.rem(sup1, 2), i1 - sup1 * g)

                compute(p)
                pltpu.async_copy(wa2.at[p], acc.at[idxb.at[q, 1, j]], sems,
                                 add=True)
                return carry

            lax.fori_loop(0, nblk, body, 0)
            wait_scatter()

        for chunk in range(4):
            pltpu.sync_copy(zh.at[pl.ds(row0, rows)], acc.at[pl.ds(row0, rows)])
            if tail:
                @pl.when(sid == _N_SUB - 1)
                def _():
                    pltpu.sync_copy(zh.at[pl.ds(_N_SUB * rows, tail)],
                                    acc.at[pl.ds(_N_SUB * rows, tail)])
            plsc.subcore_barrier()
            if chunk == 0:
                def issue0(i, p, q, j):
                    pltpu.async_copy(w0h.at[wid, i], wa2.at[p], semd)
                    pltpu.async_copy(x0h.at[idxb.at[q, 0, j]], wb2.at[p], semd)

                def waitd0(i, p):
                    pltpu.make_async_copy(w0h.at[wid, 0], wa2.at[p], semd).wait()
                    pltpu.make_async_copy(x0h.at[idxb.at[0, 0, 0]], wb2.at[p],
                                          semd).wait()

                def compute0(p):
                    def row(bb, cc):
                        for kk in range(nchunk):
                            sl = pl.ds(kk * 16, 16)
                            wa2[p, bb, sl] = wa2[p, bb, sl] * wb2[p, bb, sl]
                        return cc

                    lax.fori_loop(0, b, row, 0)

                run_pass(issue0, waitd0, compute0)
            else:
                xyh = xytabs[chunk - 1]
                wdh = (wd0h, wd1h, wd2h)[chunk - 1]

                def issue1(i, p, q, j):
                    pltpu.async_copy(wdh.at[wid, i], wa2.at[p], semd)
                    pltpu.async_copy(w2h.at[wid, i], wb2.at[p], semd)
                    pltpu.async_copy(xyh.at[idxb.at[q, 0, j]], gxy2.at[p], semd)

                def waitd1(i, p):
                    pltpu.make_async_copy(wdh.at[wid, 0], wa2.at[p], semd).wait()
                    pltpu.make_async_copy(w2h.at[wid, 0], wb2.at[p], semd).wait()
                    pltpu.make_async_copy(xyh.at[idxb.at[0, 0, 0]], gxy2.at[p],
                                          semd).wait()

                def compute1(p):
                    def row(bb, cc):
                        for kk in range(nchunk):
                            sl = pl.ds(kk * 16, 16)
                            slr = pl.ds(h + kk * 16, 16)
                            wa2[p, bb, sl] = (wa2[p, bb, sl] * gxy2[p, bb, sl]
                                              + wb2[p, bb, sl] * gxy2[p, bb, slr])
                        return cc

                    lax.fori_loop(0, b, row, 0)

                run_pass(issue1, waitd1, compute1)
            plsc.subcore_barrier()
            pltpu.sync_copy(acc.at[pl.ds(row0, rows)],
                            outh.at[chunk, cid, pl.ds(row0, rows)])
            if tail:
                @pl.when(sid == _N_SUB - 1)
                def _():
                    pltpu.sync_copy(acc.at[pl.ds(_N_SUB * rows, tail)],
                                    outh.at[chunk, cid, pl.ds(_N_SUB * rows, tail)])
            plsc.subcore_barrier()

    return k(x0, xy0, xy1, xy2, W0, Wd0, Wd1, Wd2, W2, idx2, zeros)


def _update_body(s_ref, v_ref, ep_ref, Wv_ref, Wm1_ref, bm1_ref,
                 Wm2_ref, bm2_ref, s_out_ref, v_out_ref):
    h = s_ref.shape[1]
    s1 = s_ref[...] + jnp.clip(ep_ref[0, 0] + ep_ref[0, 1], -10000.0, 10000.0)
    v1 = [v_ref[:, k, :] + jnp.clip(ep_ref[1 + k, 0] + ep_ref[1 + k, 1],
                                    -10000.0, 10000.0) for k in range(3)]
    vl = []
    vr = []
    for k in range(3):
        vmk = jnp.dot(v1[k], Wv_ref[...], preferred_element_type=jnp.float32)
        vl.append(vmk[:, :h])
        vr.append(vmk[:, h:])
    v_norm = jnp.sqrt(vl[0] ** 2 + vl[1] ** 2 + vl[2] ** 2 + 1e-08)
    ts = jnp.concatenate([s1, v_norm], axis=-1)
    m = jnp.dot(ts, Wm1_ref[...], preferred_element_type=jnp.float32) + bm1_ref[...]
    m = jnp.dot(_silu(m), Wm2_ref[...], preferred_element_type=jnp.float32) + bm2_ref[...]
    ds2 = m[:, :h]
    dvm = m[:, h:2 * h]
    dsv = m[:, 2 * h:]
    dsv = dsv * (vl[0] * vr[0] + vl[1] * vr[1] + vl[2] * vr[2])
    s_out_ref[...] = s1 + jnp.clip(ds2 + dsv, -10000.0, 10000.0)
    dvm3 = jnp.stack([v1[k] + jnp.clip(dvm * vr[k], -10000.0, 10000.0)
                      for k in range(3)], axis=1)
    v_out_ref[...] = dvm3


def _update_block(s, v, ep, Wv, Wm1, bm1, Wm2, bm2, block=1000):
    n, h = s.shape
    grid = (n // block,)
    out_shapes = (
        jax.ShapeDtypeStruct((n, h), jnp.float32),
        jax.ShapeDtypeStruct((n, 3, h), jnp.float32),
    )
    return pl.pallas_call(
        _update_body,
        grid=grid,
        in_specs=[
            pl.BlockSpec((block, h), lambda i: (i, 0)),
            pl.BlockSpec((block, 3, h), lambda i: (i, 0, 0)),
            pl.BlockSpec((4, 2, block, h), lambda i: (0, 0, i, 0)),
            pl.BlockSpec(Wv.shape, lambda i: (0, 0)),
            pl.BlockSpec(Wm1.shape, lambda i: (0, 0)),
            pl.BlockSpec(bm1.shape, lambda i: (0,)),
            pl.BlockSpec(Wm2.shape, lambda i: (0, 0)),
            pl.BlockSpec(bm2.shape, lambda i: (0,)),
        ],
        out_specs=(
            pl.BlockSpec((block, h), lambda i: (i, 0)),
            pl.BlockSpec((block, 3, h), lambda i: (i, 0, 0)),
        ),
        out_shape=out_shapes,
    )(s, v, ep, Wv, Wm1, bm1, Wm2, bm2)


def kernel(s, v, dir_ij, Wij, senders, receivers, Wi1, bi1, Wi2, bi2, Wm1, bm1, Wm2, bm2, Wv):
    n, h = s.shape
    e = senders.shape[0]
    epw = e // _NW
    nblk = epw // _B
    x0, xy0, xy1, xy2 = _node_mlp(s, v, Wi1, bi1, Wi2, bi2)
    W0, W1, W2 = jnp.split(Wij, 3, axis=1)
    W0 = W0.reshape(_NW, nblk, _B, h)
    Wd0 = (W1 * dir_ij[:, 0:1]).reshape(_NW, nblk, _B, h)
    Wd1 = (W1 * dir_ij[:, 1:2]).reshape(_NW, nblk, _B, h)
    Wd2 = (W1 * dir_ij[:, 2:3]).reshape(_NW, nblk, _B, h)
    W2 = W2.reshape(_NW, nblk, _B, h)
    g = 10
    nsup = nblk // g
    recv4 = receivers.reshape(_NW, nsup, g, _B)
    send4 = senders.reshape(_NW, nsup, g, _B)
    idx2 = jnp.stack([recv4, send4], axis=2)
    zeros = jnp.zeros((n, h), jnp.float32)
    ep = _sc_edge_stage(x0, xy0, xy1, xy2, W0, Wd0, Wd1, Wd2, W2, idx2, zeros)
    return _update_block(s, v, ep, Wv, Wm1, bm1, Wm2, bm2)
